# baseline (speedup 1.0000x reference)
import functools

import jax
import jax.numpy as jnp
from jax import lax
from jax.experimental import pallas as pl
from jax.experimental.pallas import tpu as pltpu
from jax.experimental.pallas import tpu_sc as plsc

N = 10000
E = 320000
D = 128
K = 10

NC = 2
NS = 16
NW = NC * NS
EPW = E // NW
CH = 40
NITER = EPW // CH
NB = 5
NP = 10240
RPW = NP // NS
ZR = 32
DV = NP // 128

_mesh = functools.partial(
    plsc.VectorSubcoreMesh, core_axis_name="c", subcore_axis_name="s",
    num_cores=NC, num_subcores=NS)


def _zero_vmem_2d(buf, rows, cols):
  def body_r(r, _):
    def body_k(k, __):
      buf[r, pl.ds(k * 16, 16)] = jnp.zeros((16,), jnp.float32)
      return 0
    return lax.fori_loop(0, cols // 16, body_k, 0)
  lax.fori_loop(0, rows, body_r, 0)


def _sc_degrees(edge_flat):
  @functools.partial(
      pl.kernel,
      out_type=jax.ShapeDtypeStruct((NC, 2, DV, 128), jnp.float32),
      mesh=_mesh(),
      compiler_params=pltpu.CompilerParams(needs_layout_passes=False),
      scratch_types=[
          pltpu.VMEM_SHARED((DV, 128), jnp.float32),
          pltpu.VMEM_SHARED((DV, 128), jnp.float32),
          pltpu.VMEM((DV, 128), jnp.float32),
          pltpu.VMEM((DV, 128), jnp.float32),
          pltpu.VMEM((EPW,), jnp.int32),
          pltpu.VMEM((EPW,), jnp.int32),
          pltpu.VMEM((DV,), jnp.int32),
          pltpu.SemaphoreType.DMA,
          pltpu.SemaphoreType.DMA,
      ],
  )
  def deg_kernel(ei_hbm, out_hbm, shr_s, shr_i, acc_s, acc_i,
                 sidx_all, didx_all, rowidx, sem_a, sem_b):
    cid = lax.axis_index("c")
    sid = lax.axis_index("s")
    wid = sid * NC + cid
    e0 = wid * EPW
    cp_s = pltpu.async_copy(ei_hbm.at[pl.ds(e0, EPW)], sidx_all, sem_a)
    cp_d = pltpu.async_copy(ei_hbm.at[pl.ds(E + e0, EPW)], didx_all, sem_b)
    _zero_vmem_2d(acc_s, DV, 128)
    _zero_vmem_2d(acc_i, DV, 128)
    pltpu.sync_copy(acc_s.at[pl.ds(0, DV // NS)],
                    shr_s.at[pl.ds(sid * (DV // NS), DV // NS)])
    pltpu.sync_copy(acc_i.at[pl.ds(0, DV // NS)],
                    shr_i.at[pl.ds(sid * (DV // NS), DV // NS)])
    def mk_row(j, _):
      rowidx[pl.ds(j * 16, 16)] = lax.iota(jnp.int32, 16) + j * 16
      return 0
    lax.fori_loop(0, DV // 16, mk_row, 0)
    cp_s.wait()
    cp_d.wait()
    plsc.subcore_barrier()

    ones16 = jnp.ones((16,), jnp.float32)

    def count(v, _):
      ns = sidx_all[pl.ds(v * 16, 16)]
      plsc.addupdate_scatter(
          acc_s, [lax.shift_right_logical(ns, 7), lax.bitwise_and(ns, 127)],
          ones16)
      nd = didx_all[pl.ds(v * 16, 16)]
      plsc.addupdate_scatter(
          acc_i, [lax.shift_right_logical(nd, 7), lax.bitwise_and(nd, 127)],
          ones16)
      return 0

    lax.fori_loop(0, EPW // 16, count, 0)
    pltpu.sync_copy(acc_s, shr_s.at[rowidx], add=True)
    pltpu.sync_copy(acc_i, shr_i.at[rowidx], add=True)
    plsc.subcore_barrier()

    r0 = jnp.minimum(sid, 9) * 8
    pltpu.sync_copy(shr_s.at[pl.ds(r0, 8)], out_hbm.at[cid, 0, pl.ds(r0, 8)])
    pltpu.sync_copy(shr_i.at[pl.ds(r0, 8)], out_hbm.at[cid, 1, pl.ds(r0, 8)])

  return deg_kernel(edge_flat)


def _sc_spmm(v, edge_flat):
  @functools.partial(
      pl.kernel,
      out_type=jax.ShapeDtypeStruct((NC, NP, D), jnp.float32),
      mesh=_mesh(),
      scratch_types=[
          pltpu.VMEM_SHARED((NP, D), jnp.float32),
          pltpu.VMEM((ZR, D), jnp.float32),
          [pltpu.VMEM((CH, D), jnp.float32)] * NB,
          pltpu.VMEM((EPW,), jnp.int32),
          [pltpu.VMEM((CH,), jnp.int32)] * NB,
          [pltpu.SemaphoreType.DMA] * NB,
          [pltpu.SemaphoreType.DMA] * NB,
          [pltpu.SemaphoreType.DMA] * NB,
          pltpu.SemaphoreType.DMA,
      ],
  )
  def spmm_kernel(v_hbm, ei_hbm, out_hbm,
                  acc, zbuf, rows, sidx_all, dchunk,
                  gsem, dsem, ssem, sem_a):
    cid = lax.axis_index("c")
    sid = lax.axis_index("s")
    wid = sid * NC + cid
    r0 = sid * RPW
    e0 = wid * EPW
    cp_s = pltpu.async_copy(ei_hbm.at[pl.ds(e0, EPW)], sidx_all, sem_a)
    _zero_vmem_2d(zbuf, ZR, D)
    for k in range(RPW // ZR):
      pltpu.sync_copy(zbuf, acc.at[pl.ds(r0 + k * ZR, ZR)])
    cp_s.wait()
    plsc.subcore_barrier()

    AH = 3

    def load(i, b, wait_scatter):
      if wait_scatter:
        pltpu.make_async_copy(rows[b], acc.at[dchunk[b]], ssem[b]).wait()
      pltpu.async_copy(
          v_hbm.at[sidx_all.at[pl.ds(i * CH, CH)]], rows[b], gsem[b])
      pltpu.async_copy(ei_hbm.at[pl.ds(E + e0 + i * CH, CH)], dchunk[b],
                       dsem[b])

    def consume(b):
      pltpu.make_async_copy(
          v_hbm.at[sidx_all.at[pl.ds(0, CH)]], rows[b], gsem[b]).wait()
      pltpu.make_async_copy(ei_hbm.at[pl.ds(E + e0, CH)], dchunk[b],
                            dsem[b]).wait()
      pltpu.async_copy(rows[b], acc.at[dchunk[b]], ssem[b], add=True)

    for i in range(AH):
      load(i, i, False)
    for i in range(NB):
      consume(i % NB)
      load(i + AH, (i + AH) % NB, wait_scatter=(i >= NB - AH))

    def outer(g, _):
      i0 = NB + g * NB
      for b in range(NB):
        consume(b)
        load(i0 + b + AH, (b + AH) % NB, True)
      return 0

    lax.fori_loop(0, NITER // NB - 2, outer, 0)
    for b in range(NB):
      consume(b)
      if b < NB - AH:
        load(NITER - NB + b + AH, (b + AH) % NB, True)
    for b in range(NB):
      pltpu.make_async_copy(rows[b], acc.at[dchunk[b]], ssem[b]).wait()

    plsc.subcore_barrier()
    for k in range(RPW // ZR):
      pltpu.sync_copy(acc.at[pl.ds(r0 + k * ZR, ZR)],
                      out_hbm.at[cid, pl.ds(r0 + k * ZR, ZR)])

  return spmm_kernel(v, edge_flat)


BN = 400
NG = N // BN


def _row_spec():
  return pl.BlockSpec((BN, D), lambda i: (i, 0))


def _full_spec(shape):
  nd = len(shape)
  return pl.BlockSpec(shape, lambda i: (0,) * nd)


def _vec_spec():
  return pl.BlockSpec((1, 1, BN), lambda i: (i, 0, 0))


def _part_spec():
  return pl.BlockSpec((NC, BN, D), lambda i: (0, i, 0))


def _tc_deg_inv(degp):
  def body(degp_ref, dinv_ref):
    od = degp_ref[0, 0] + degp_ref[1, 0]
    idg = degp_ref[0, 1] + degp_ref[1, 1]
    dinv_ref[0] = lax.rsqrt(jnp.maximum(od, 1.0))
    dinv_ref[1] = lax.rsqrt(jnp.maximum(idg, 1.0))

  return pl.pallas_call(
      body,
      grid=(1,),
      in_specs=[_full_spec((NC, 2, DV, 128))],
      out_specs=_full_spec((2, DV, 128)),
      out_shape=jax.ShapeDtypeStruct((2, DV, 128), jnp.float32),
  )(degp)


def _tc_prep(x, oinv3, bmw3, bmb):
  def body(x_ref, oinv_ref, bmw_ref, bmb_ref, xs_ref, gene_ref):
    i = pl.program_id(0)
    xb = x_ref[...]
    xs_ref[...] = xb * oinv_ref[0, 0, :][:, None]
    w = bmw_ref[0, 0, :][:, None]
    part = jnp.sum(xb * w, axis=0, keepdims=True)

    @pl.when(i == 0)
    def _():
      gene_ref[...] = jnp.zeros((1, D), jnp.float32)

    gene_ref[...] += part

    @pl.when(i == NG - 1)
    def _():
      gene_ref[...] += bmb_ref[0, 0]

  return pl.pallas_call(
      body,
      grid=(NG,),
      in_specs=[_row_spec(), _vec_spec(), _vec_spec(), _full_spec((1, 1))],
      out_specs=[_row_spec(), pl.BlockSpec((1, D), lambda i: (0, 0))],
      out_shape=[
          jax.ShapeDtypeStruct((N, D), jnp.float32),
          jax.ShapeDtypeStruct((1, D), jnp.float32),
      ],
  )(x, oinv3, bmw3, bmb)


def _tc_layer1(p, iinv3, oinv3, w0, b0):
  def body(p_ref, iinv_ref, oinv_ref, w_ref, b_ref, out_ref):
    agg = (p_ref[0] + p_ref[1]) * iinv_ref[0, 0, :][:, None]
    h = jnp.dot(agg, w_ref[...], preferred_element_type=jnp.float32)
    h = jnp.maximum(h + b_ref[...], 0.0)
    out_ref[...] = h * oinv_ref[0, 0, :][:, None]

  return pl.pallas_call(
      body,
      grid=(NG,),
      in_specs=[_part_spec(), _vec_spec(), _vec_spec(),
                _full_spec((D, D)), _full_spec((1, D))],
      out_specs=_row_spec(),
      out_shape=jax.ShapeDtypeStruct((N, D), jnp.float32),
  )(p, iinv3, oinv3, w0, b0)


def _tc_layer2(p, iinv3, oinv3, w1, b1, pc, pr, cwp, cbp, lbl3):
  def body(p_ref, iinv_ref, oinv_ref, w_ref, b_ref, pc_ref, pr_ref, cw_ref,
           cb_ref, lbl_ref, pred_ref, rrs_ref, closs_ref):
    i = pl.program_id(0)
    agg = (p_ref[0] + p_ref[1]) * iinv_ref[0, 0, :][:, None]
    enc = jnp.dot(agg, w_ref[...], preferred_element_type=jnp.float32)
    enc = jnp.maximum(enc + b_ref[...], 0.0)
    rc = jnp.dot(enc, pc_ref[...], preferred_element_type=jnp.float32)
    pred = jnp.dot(rc, cw_ref[...], preferred_element_type=jnp.float32)
    pred = pred + cb_ref[...]
    pred_ref[...] = pred
    rr = jnp.dot(enc, pr_ref[...], preferred_element_type=jnp.float32)
    rrs_ref[...] = rr * oinv_ref[0, 0, :][:, None]
    lanes = lax.broadcasted_iota(jnp.int32, (BN, D), 1)
    valid = lanes < K
    neg = jnp.float32(-1e30)
    m = jnp.max(jnp.where(valid, pred, neg), axis=1, keepdims=True)
    ex = jnp.where(valid, jnp.exp(pred - m), 0.0)
    lse = jnp.log(jnp.sum(ex, axis=1, keepdims=True)) + m
    logp = pred - lse
    onehot = lanes == lbl_ref[0, 0, :][:, None]
    logpt = jnp.sum(jnp.where(onehot, logp, 0.0), axis=1)
    pt = jnp.exp(logpt)
    contrib = -((1.0 - pt) ** 2) * logpt

    @pl.when(i == 0)
    def _():
      closs_ref[0, 0] = 0.0

    closs_ref[0, 0] += jnp.sum(contrib)

    @pl.when(i == NG - 1)
    def _():
      closs_ref[0, 0] *= jnp.float32(1.0 / N)

  return pl.pallas_call(
      body,
      grid=(NG,),
      in_specs=[_part_spec(), _vec_spec(), _vec_spec(),
                _full_spec((D, D)), _full_spec((1, D)),
                _full_spec((D, D)), _full_spec((D, D)),
                _full_spec((D, D)), _full_spec((1, D)),
                _vec_spec()],
      out_specs=[_row_spec(), _row_spec(),
                 pl.BlockSpec(memory_space=pltpu.SMEM)],
      out_shape=[
          jax.ShapeDtypeStruct((N, D), jnp.float32),
          jax.ShapeDtypeStruct((N, D), jnp.float32),
          jax.ShapeDtypeStruct((1, 1), jnp.float32),
      ],
  )(p, iinv3, oinv3, w1, b1, pc, pr, cwp, cbp, lbl3)


def _tc_decoder(p, iinv3, decw, decb, gene, x):
  def body(p_ref, iinv_ref, w_ref, b_ref, g_ref, x_ref, rloss_ref):
    i = pl.program_id(0)
    agg = (p_ref[0] + p_ref[1]) * iinv_ref[0, 0, :][:, None]
    xr = jnp.dot(agg, w_ref[...], preferred_element_type=jnp.float32)
    xr = xr + b_ref[...] + g_ref[...]
    xb = x_ref[...]
    nx = jnp.sqrt(jnp.sum(xb * xb, axis=1))
    ny = jnp.sqrt(jnp.sum(xr * xr, axis=1))
    dt = jnp.sum(xb * xr, axis=1)
    cos = dt / (jnp.maximum(nx, 1e-12) * jnp.maximum(ny, 1e-12))
    contrib = (1.0 - cos) ** 2

    @pl.when(i == 0)
    def _():
      rloss_ref[0, 0] = 0.0

    rloss_ref[0, 0] += jnp.sum(contrib)

    @pl.when(i == NG - 1)
    def _():
      rloss_ref[0, 0] *= jnp.float32(1.0 / N)

  return pl.pallas_call(
      body,
      grid=(NG,),
      in_specs=[_part_spec(), _vec_spec(), _full_spec((D, D)),
                _full_spec((1, D)), pl.BlockSpec((1, D), lambda i: (0, 0)),
                _row_spec()],
      out_specs=pl.BlockSpec(memory_space=pltpu.SMEM),
      out_shape=jax.ShapeDtypeStruct((1, 1), jnp.float32),
  )(p, iinv3, decw, decb, gene, x)


def kernel(x, edge_index, label, enc_W0, enc_b0, enc_W1, enc_b1,
           proj_rec_W, proj_cls_W, cls_W, cls_b, dec_W, dec_b, bm_W, bm_b):
  edge_flat = edge_index.reshape(-1)

  degp = _sc_degrees(edge_flat)
  dinv = _tc_deg_inv(degp)
  dinv3 = dinv.reshape(2, NP)[:, :N].reshape(2, NG, 1, BN)
  oinv3 = dinv3[0]
  iinv3 = dinv3[1]

  bmw3 = bm_W.reshape(NG, 1, BN)
  bmb = bm_b.reshape(1, 1)
  xs, gene = _tc_prep(x, oinv3, bmw3, bmb)

  p1 = _sc_spmm(xs, edge_flat)
  hs = _tc_layer1(p1, iinv3, oinv3, enc_W0, enc_b0.reshape(1, D))

  p2 = _sc_spmm(hs, edge_flat)
  cwp = jnp.zeros((D, D), jnp.float32).at[:, :K].set(cls_W)
  cbp = jnp.zeros((1, D), jnp.float32).at[0, :K].set(cls_b)
  lbl3 = label.reshape(NG, 1, BN)
  pred_pad, rrs, closs = _tc_layer2(
      p2, iinv3, oinv3, enc_W1, enc_b1.reshape(1, D), proj_cls_W, proj_rec_W,
      cwp, cbp, lbl3)

  p3 = _sc_spmm(rrs, edge_flat)
  rloss = _tc_decoder(p3, iinv3, dec_W, dec_b.reshape(1, D), gene, x)

  return (rloss[0, 0], closs[0, 0], pred_pad[:, :K])


# speedup vs baseline: 1.1194x; 1.1194x over previous
import functools

import jax
import jax.numpy as jnp
from jax import lax
from jax.experimental import pallas as pl
from jax.experimental.pallas import tpu as pltpu
from jax.experimental.pallas import tpu_sc as plsc

N = 10000
E = 320000
D = 128
K = 10

NC = 2
NS = 16
NW = NC * NS
EPW = E // NW
CH = 40
NITER = EPW // CH
NB = 5
NP = 10240
RPW = NP // NS
ZR = 32
DV = NP // 128

_mesh = functools.partial(
    plsc.VectorSubcoreMesh, core_axis_name="c", subcore_axis_name="s",
    num_cores=NC, num_subcores=NS)


def _zero_vmem_2d(buf, rows, cols):
  def body_r(r, _):
    def body_k(k, __):
      buf[r, pl.ds(k * 16, 16)] = jnp.zeros((16,), jnp.float32)
      return 0
    return lax.fori_loop(0, cols // 16, body_k, 0)
  lax.fori_loop(0, rows, body_r, 0)


def _sc_degrees(edge_flat):
  @functools.partial(
      pl.kernel,
      out_type=jax.ShapeDtypeStruct((NC, 2, DV, 128), jnp.float32),
      mesh=_mesh(),
      compiler_params=pltpu.CompilerParams(needs_layout_passes=False),
      scratch_types=[
          pltpu.VMEM_SHARED((DV, 128), jnp.float32),
          pltpu.VMEM_SHARED((DV, 128), jnp.float32),
          pltpu.VMEM((DV, 128), jnp.float32),
          pltpu.VMEM((DV, 128), jnp.float32),
          pltpu.VMEM((EPW,), jnp.int32),
          pltpu.VMEM((EPW,), jnp.int32),
          pltpu.VMEM((DV,), jnp.int32),
          pltpu.SemaphoreType.DMA,
          pltpu.SemaphoreType.DMA,
      ],
  )
  def deg_kernel(ei_hbm, out_hbm, shr_s, shr_i, acc_s, acc_i,
                 sidx_all, didx_all, rowidx, sem_a, sem_b):
    cid = lax.axis_index("c")
    sid = lax.axis_index("s")
    wid = sid * NC + cid
    e0 = wid * EPW
    cp_s = pltpu.async_copy(ei_hbm.at[pl.ds(e0, EPW)], sidx_all, sem_a)
    cp_d = pltpu.async_copy(ei_hbm.at[pl.ds(E + e0, EPW)], didx_all, sem_b)
    _zero_vmem_2d(acc_s, DV, 128)
    _zero_vmem_2d(acc_i, DV, 128)
    pltpu.sync_copy(acc_s.at[pl.ds(0, DV // NS)],
                    shr_s.at[pl.ds(sid * (DV // NS), DV // NS)])
    pltpu.sync_copy(acc_i.at[pl.ds(0, DV // NS)],
                    shr_i.at[pl.ds(sid * (DV // NS), DV // NS)])
    def mk_row(j, _):
      rowidx[pl.ds(j * 16, 16)] = lax.iota(jnp.int32, 16) + j * 16
      return 0
    lax.fori_loop(0, DV // 16, mk_row, 0)
    cp_s.wait()
    cp_d.wait()
    plsc.subcore_barrier()

    ones16 = jnp.ones((16,), jnp.float32)

    def count(v, _):
      ns = sidx_all[pl.ds(v * 16, 16)]
      plsc.addupdate_scatter(
          acc_s, [lax.shift_right_logical(ns, 7), lax.bitwise_and(ns, 127)],
          ones16)
      nd = didx_all[pl.ds(v * 16, 16)]
      plsc.addupdate_scatter(
          acc_i, [lax.shift_right_logical(nd, 7), lax.bitwise_and(nd, 127)],
          ones16)
      return 0

    lax.fori_loop(0, EPW // 16, count, 0)
    pltpu.sync_copy(acc_s, shr_s.at[rowidx], add=True)
    pltpu.sync_copy(acc_i, shr_i.at[rowidx], add=True)
    plsc.subcore_barrier()

    r0 = jnp.minimum(sid, 9) * 8
    pltpu.sync_copy(shr_s.at[pl.ds(r0, 8)], out_hbm.at[cid, 0, pl.ds(r0, 8)])
    pltpu.sync_copy(shr_i.at[pl.ds(r0, 8)], out_hbm.at[cid, 1, pl.ds(r0, 8)])

  return deg_kernel(edge_flat)


def _sc_spmm(v, edge_flat):
  @functools.partial(
      pl.kernel,
      out_type=jax.ShapeDtypeStruct((NC, NP, D), jnp.float32),
      mesh=_mesh(),
      scratch_types=[
          pltpu.VMEM_SHARED((NP, D), jnp.float32),
          pltpu.VMEM((ZR, D), jnp.float32),
          [pltpu.VMEM((CH, D), jnp.float32)] * NB,
          pltpu.VMEM((EPW,), jnp.int32),
          [pltpu.VMEM((CH,), jnp.int32)] * NB,
          [pltpu.SemaphoreType.DMA] * NB,
          [pltpu.SemaphoreType.DMA] * NB,
          pltpu.SemaphoreType.DMA,
      ],
  )
  def spmm_kernel(v_hbm, ei_hbm, out_hbm,
                  acc, zbuf, rows, sidx_all, dchunk,
                  gsem, dsem, sem_a):
    cid = lax.axis_index("c")
    sid = lax.axis_index("s")
    wid = sid * NC + cid
    r0 = sid * RPW
    e0 = wid * EPW
    cp_s = pltpu.async_copy(ei_hbm.at[pl.ds(e0, EPW)], sidx_all, sem_a)
    _zero_vmem_2d(zbuf, ZR, D)
    for k in range(RPW // ZR):
      pltpu.sync_copy(zbuf, acc.at[pl.ds(r0 + k * ZR, ZR)])
    cp_s.wait()
    plsc.subcore_barrier()

    def start(i, b):
      pltpu.async_copy(
          v_hbm.at[sidx_all.at[pl.ds(i * CH, CH)]], rows[b], gsem[b])
      pltpu.async_copy(ei_hbm.at[pl.ds(E + e0 + i * CH, CH)], dchunk[b],
                       dsem[b])

    def consume(b):
      pltpu.make_async_copy(
          v_hbm.at[sidx_all.at[pl.ds(0, CH)]], rows[b], gsem[b]).wait()
      pltpu.make_async_copy(ei_hbm.at[pl.ds(E + e0, CH)], dchunk[b],
                            dsem[b]).wait()
      pltpu.sync_copy(rows[b], acc.at[dchunk[b]], add=True)

    for b in range(NB):
      start(b, b)

    def outer(g, _):
      i0 = g * NB
      for b in range(NB):
        consume(b)
        start(i0 + b + NB, b)
      return 0

    lax.fori_loop(0, NITER // NB - 1, outer, 0)
    for b in range(NB):
      consume(b)

    plsc.subcore_barrier()
    for k in range(RPW // ZR):
      pltpu.sync_copy(acc.at[pl.ds(r0 + k * ZR, ZR)],
                      out_hbm.at[cid, pl.ds(r0 + k * ZR, ZR)])

  return spmm_kernel(v, edge_flat)


BN = 400
NG = N // BN


def _row_spec():
  return pl.BlockSpec((BN, D), lambda i: (i, 0))


def _full_spec(shape):
  nd = len(shape)
  return pl.BlockSpec(shape, lambda i: (0,) * nd)


def _vec_spec():
  return pl.BlockSpec((1, 1, BN), lambda i: (i, 0, 0))


def _part_spec():
  return pl.BlockSpec((NC, BN, D), lambda i: (0, i, 0))


def _tc_deg_inv(degp):
  def body(degp_ref, dinv_ref):
    od = degp_ref[0, 0] + degp_ref[1, 0]
    idg = degp_ref[0, 1] + degp_ref[1, 1]
    dinv_ref[0] = lax.rsqrt(jnp.maximum(od, 1.0))
    dinv_ref[1] = lax.rsqrt(jnp.maximum(idg, 1.0))

  return pl.pallas_call(
      body,
      grid=(1,),
      in_specs=[_full_spec((NC, 2, DV, 128))],
      out_specs=_full_spec((2, DV, 128)),
      out_shape=jax.ShapeDtypeStruct((2, DV, 128), jnp.float32),
  )(degp)


def _tc_prep(x, oinv3, bmw3, bmb):
  def body(x_ref, oinv_ref, bmw_ref, bmb_ref, xs_ref, gene_ref):
    i = pl.program_id(0)
    xb = x_ref[...]
    xs_ref[...] = xb * oinv_ref[0, 0, :][:, None]
    w = bmw_ref[0, 0, :][:, None]
    part = jnp.sum(xb * w, axis=0, keepdims=True)

    @pl.when(i == 0)
    def _():
      gene_ref[...] = jnp.zeros((1, D), jnp.float32)

    gene_ref[...] += part

    @pl.when(i == NG - 1)
    def _():
      gene_ref[...] += bmb_ref[0, 0]

  return pl.pallas_call(
      body,
      grid=(NG,),
      in_specs=[_row_spec(), _vec_spec(), _vec_spec(), _full_spec((1, 1))],
      out_specs=[_row_spec(), pl.BlockSpec((1, D), lambda i: (0, 0))],
      out_shape=[
          jax.ShapeDtypeStruct((N, D), jnp.float32),
          jax.ShapeDtypeStruct((1, D), jnp.float32),
      ],
  )(x, oinv3, bmw3, bmb)


def _tc_layer1(p, iinv3, oinv3, w0, b0):
  def body(p_ref, iinv_ref, oinv_ref, w_ref, b_ref, out_ref):
    agg = (p_ref[0] + p_ref[1]) * iinv_ref[0, 0, :][:, None]
    h = jnp.dot(agg, w_ref[...], preferred_element_type=jnp.float32)
    h = jnp.maximum(h + b_ref[...], 0.0)
    out_ref[...] = h * oinv_ref[0, 0, :][:, None]

  return pl.pallas_call(
      body,
      grid=(NG,),
      in_specs=[_part_spec(), _vec_spec(), _vec_spec(),
                _full_spec((D, D)), _full_spec((1, D))],
      out_specs=_row_spec(),
      out_shape=jax.ShapeDtypeStruct((N, D), jnp.float32),
  )(p, iinv3, oinv3, w0, b0)


def _tc_layer2a(p, iinv3, oinv3, w1, b1, pr):
  def body(p_ref, iinv_ref, oinv_ref, w_ref, b_ref, pr_ref, rrs_ref,
           enc_ref):
    agg = (p_ref[0] + p_ref[1]) * iinv_ref[0, 0, :][:, None]
    enc = jnp.dot(agg, w_ref[...], preferred_element_type=jnp.float32)
    enc = jnp.maximum(enc + b_ref[...], 0.0)
    enc_ref[...] = enc
    rr = jnp.dot(enc, pr_ref[...], preferred_element_type=jnp.float32)
    rrs_ref[...] = rr * oinv_ref[0, 0, :][:, None]

  return pl.pallas_call(
      body,
      grid=(NG,),
      in_specs=[_part_spec(), _vec_spec(), _vec_spec(),
                _full_spec((D, D)), _full_spec((1, D)), _full_spec((D, D))],
      out_specs=[_row_spec(), _row_spec()],
      out_shape=[
          jax.ShapeDtypeStruct((N, D), jnp.float32),
          jax.ShapeDtypeStruct((N, D), jnp.float32),
      ],
  )(p, iinv3, oinv3, w1, b1, pr)


def _tc_layer2b(enc, pc, cwp, cbp, lbl3):
  def body(enc_ref, pc_ref, cw_ref, cb_ref, lbl_ref, pred_ref, closs_ref):
    i = pl.program_id(0)
    enc = enc_ref[...]
    rc = jnp.dot(enc, pc_ref[...], preferred_element_type=jnp.float32)
    pred = jnp.dot(rc, cw_ref[...], preferred_element_type=jnp.float32)
    pred = pred + cb_ref[...]
    pred_ref[...] = pred
    lanes = lax.broadcasted_iota(jnp.int32, (BN, D), 1)
    valid = lanes < K
    neg = jnp.float32(-1e30)
    m = jnp.max(jnp.where(valid, pred, neg), axis=1, keepdims=True)
    ex = jnp.where(valid, jnp.exp(pred - m), 0.0)
    lse = jnp.log(jnp.sum(ex, axis=1, keepdims=True)) + m
    logp = pred - lse
    onehot = lanes == lbl_ref[0, 0, :][:, None]
    logpt = jnp.sum(jnp.where(onehot, logp, 0.0), axis=1)
    pt = jnp.exp(logpt)
    contrib = -((1.0 - pt) ** 2) * logpt

    @pl.when(i == 0)
    def _():
      closs_ref[0, 0] = 0.0

    closs_ref[0, 0] += jnp.sum(contrib)

    @pl.when(i == NG - 1)
    def _():
      closs_ref[0, 0] *= jnp.float32(1.0 / N)

  return pl.pallas_call(
      body,
      grid=(NG,),
      in_specs=[_row_spec(), _full_spec((D, D)),
                _full_spec((D, D)), _full_spec((1, D)),
                _vec_spec()],
      out_specs=[_row_spec(), pl.BlockSpec(memory_space=pltpu.SMEM)],
      out_shape=[
          jax.ShapeDtypeStruct((N, D), jnp.float32),
          jax.ShapeDtypeStruct((1, 1), jnp.float32),
      ],
  )(enc, pc, cwp, cbp, lbl3)


def _tc_decoder(p, iinv3, decw, decb, gene, x):
  def body(p_ref, iinv_ref, w_ref, b_ref, g_ref, x_ref, rloss_ref):
    i = pl.program_id(0)
    agg = (p_ref[0] + p_ref[1]) * iinv_ref[0, 0, :][:, None]
    xr = jnp.dot(agg, w_ref[...], preferred_element_type=jnp.float32)
    xr = xr + b_ref[...] + g_ref[...]
    xb = x_ref[...]
    nx = jnp.sqrt(jnp.sum(xb * xb, axis=1))
    ny = jnp.sqrt(jnp.sum(xr * xr, axis=1))
    dt = jnp.sum(xb * xr, axis=1)
    cos = dt / (jnp.maximum(nx, 1e-12) * jnp.maximum(ny, 1e-12))
    contrib = (1.0 - cos) ** 2

    @pl.when(i == 0)
    def _():
      rloss_ref[0, 0] = 0.0

    rloss_ref[0, 0] += jnp.sum(contrib)

    @pl.when(i == NG - 1)
    def _():
      rloss_ref[0, 0] *= jnp.float32(1.0 / N)

  return pl.pallas_call(
      body,
      grid=(NG,),
      in_specs=[_part_spec(), _vec_spec(), _full_spec((D, D)),
                _full_spec((1, D)), pl.BlockSpec((1, D), lambda i: (0, 0)),
                _row_spec()],
      out_specs=pl.BlockSpec(memory_space=pltpu.SMEM),
      out_shape=jax.ShapeDtypeStruct((1, 1), jnp.float32),
  )(p, iinv3, decw, decb, gene, x)


def kernel(x, edge_index, label, enc_W0, enc_b0, enc_W1, enc_b1,
           proj_rec_W, proj_cls_W, cls_W, cls_b, dec_W, dec_b, bm_W, bm_b):
  edge_flat = edge_index.reshape(-1)

  degp = _sc_degrees(edge_flat)
  dinv = _tc_deg_inv(degp)
  dinv3 = dinv.reshape(2, NP)[:, :N].reshape(2, NG, 1, BN)
  oinv3 = dinv3[0]
  iinv3 = dinv3[1]

  bmw3 = bm_W.reshape(NG, 1, BN)
  bmb = bm_b.reshape(1, 1)
  xs, gene = _tc_prep(x, oinv3, bmw3, bmb)

  p1 = _sc_spmm(xs, edge_flat)
  hs = _tc_layer1(p1, iinv3, oinv3, enc_W0, enc_b0.reshape(1, D))

  p2 = _sc_spmm(hs, edge_flat)
  cwp = jnp.zeros((D, D), jnp.float32).at[:, :K].set(cls_W)
  cbp = jnp.zeros((1, D), jnp.float32).at[0, :K].set(cls_b)
  lbl3 = label.reshape(NG, 1, BN)
  rrs, enc = _tc_layer2a(p2, iinv3, oinv3, enc_W1, enc_b1.reshape(1, D),
                         proj_rec_W)

  p3 = _sc_spmm(rrs, edge_flat)
  pred_pad, closs = _tc_layer2b(enc, proj_cls_W, cwp, cbp, lbl3)
  rloss = _tc_decoder(p3, iinv3, dec_W, dec_b.reshape(1, D), gene, x)

  return (rloss[0, 0], closs[0, 0], pred_pad[:, :K])
